# baseline (speedup 1.0000x reference)
import jax
import jax.numpy as jnp
from jax import lax
from jax.experimental import pallas as pl
from jax.experimental.pallas import tpu as pltpu
from jax.experimental.pallas import tpu_sc as plsc

_NC, _NS, _L = 2, 16, 16
_NW = _NC * _NS
_B = 16384
_BPW = _B // _NW
_D = 32
_CH = 128
_NCH = _BPW // _CH
_GROUPS = _BPW // _L
_BIAS = 3.5


def _sc_body(user_h, item_h, label_h, utab_h, itab_h,
             pred_h, rmse_h, part_h,
             idxu, idxi, urows, irows, labv, predv, rmsev, sqv, sem):
    wid = lax.axis_index("s") * _NC + lax.axis_index("c")

    pltpu.sync_copy(user_h.at[wid], idxu)
    pltpu.sync_copy(item_h.at[wid], idxi)
    pltpu.sync_copy(label_h.at[wid], labv)

    copies = []
    for j in range(_NCH):
        copies.append(pltpu.async_copy(
            utab_h.at[idxu.at[j]], urows.at[pl.ds(j * _CH, _CH)], sem))
        copies.append(pltpu.async_copy(
            itab_h.at[idxi.at[j]], irows.at[pl.ds(j * _CH, _CH)], sem))
    for c in copies:
        c.wait()

    lane = lax.iota(jnp.int32, _L)

    def g_body(g, sq_acc):
        rows = g * _L + lane
        acc = jnp.zeros((_L,), jnp.float32)
        for d in range(_D):
            dcol = jnp.full((_L,), d, jnp.int32)
            u16 = plsc.load_gather(urows, [rows, dcol])
            v16 = plsc.load_gather(irows, [rows, dcol])
            acc = acc + u16 * v16
        pred16 = acc + _BIAS
        base = pl.multiple_of(g * _L, _L)
        predv[pl.ds(base, _L)] = pred16
        diff = pred16 - labv[pl.ds(base, _L)]
        rmsev[pl.ds(base, _L)] = jnp.abs(diff)
        return sq_acc + diff * diff

    sq = lax.fori_loop(0, _GROUPS, g_body, jnp.zeros((_L,), jnp.float32))
    sqv[...] = sq

    pltpu.sync_copy(predv, pred_h.at[wid])
    pltpu.sync_copy(rmsev, rmse_h.at[wid])
    pltpu.sync_copy(sqv, part_h.at[wid])


def _obj_body(p_ref, o_ref):
    o_ref[0, 0] = jnp.sum(p_ref[...]) * (1.0 / _B)


def kernel(user, item, label, user_table, item_table):
    f32 = jnp.float32
    sc_fn = pl.kernel(
        _sc_body,
        out_type=(
            jax.ShapeDtypeStruct((_NW, _BPW), f32),
            jax.ShapeDtypeStruct((_NW, _BPW), f32),
            jax.ShapeDtypeStruct((_NW, _L), f32),
        ),
        mesh=plsc.VectorSubcoreMesh(core_axis_name="c", subcore_axis_name="s"),
        compiler_params=pltpu.CompilerParams(
            needs_layout_passes=False, use_tc_tiling_on_sc=False),
        scratch_types=[
            pltpu.VMEM((_NCH, _CH), jnp.int32),
            pltpu.VMEM((_NCH, _CH), jnp.int32),
            pltpu.VMEM((_BPW, _D), f32),
            pltpu.VMEM((_BPW, _D), f32),
            pltpu.VMEM((_BPW,), f32),
            pltpu.VMEM((_BPW,), f32),
            pltpu.VMEM((_BPW,), f32),
            pltpu.VMEM((_L,), f32),
            pltpu.SemaphoreType.DMA,
        ],
    )
    pred2d, rmse2d, part = sc_fn(
        user.reshape(_NW, _NCH, _CH),
        item.reshape(_NW, _NCH, _CH),
        label.reshape(_NW, _BPW),
        user_table,
        item_table,
    )

    obj2 = pl.pallas_call(
        _obj_body,
        out_shape=jax.ShapeDtypeStruct((1, 1), f32),
        out_specs=pl.BlockSpec(memory_space=pltpu.SMEM),
    )(part)

    return (pred2d.reshape(-1), obj2[0, 0], rmse2d.reshape(-1))
